# Initial kernel scaffold; baseline (speedup 1.0000x reference)
import jax
import jax.numpy as jnp
from jax.experimental import pallas as pl


def kernel(x, table):
    raise NotImplementedError("write your pallas kernel here")


# speedup vs baseline: 1.1603x; 1.1603x over previous
import functools

import jax
import jax.numpy as jnp
from jax import lax
from jax.experimental import pallas as pl
from jax.experimental.pallas import tpu as pltpu
from jax.experimental.pallas import tpu_sc as plsc

NUMS = 1000000
DIMS = 32
BATCH = 16384
HIST = 50
B = BATCH * HIST

NUM_WORKERS = 32
B_PER_W = B // NUM_WORKERS
CHUNK = 3200
N_CHUNKS = B_PER_W // CHUNK


def _sc_gather(table, idx):
    mesh = plsc.VectorSubcoreMesh(core_axis_name="c", subcore_axis_name="s")

    @functools.partial(
        pl.kernel,
        mesh=mesh,
        out_type=jax.ShapeDtypeStruct((B, DIMS), jnp.float32),
        scratch_types=[
            pltpu.VMEM((CHUNK,), jnp.int32),
            pltpu.VMEM((CHUNK, DIMS), jnp.float32),
            pltpu.SemaphoreType.DMA,
        ],
        compiler_params=pltpu.CompilerParams(use_tc_tiling_on_sc=False),
    )
    def k(table_hbm, idx_hbm, out_hbm, idx_v, rows_v, sem):
        wid = lax.axis_index("s") * 2 + lax.axis_index("c")
        w_base = wid * B_PER_W

        def body(i, _):
            base = w_base + i * CHUNK
            pltpu.sync_copy(idx_hbm.at[pl.ds(base, CHUNK)], idx_v)
            pltpu.async_copy(table_hbm.at[idx_v], rows_v, sem).wait()
            pltpu.sync_copy(rows_v, out_hbm.at[pl.ds(base, CHUNK)])
            return 0

        lax.fori_loop(0, N_CHUNKS, body, 0)

    return k(table, idx)


def kernel(x, table):
    idx = x.reshape(B)
    out = _sc_gather(table, idx)
    return out.reshape(BATCH, HIST, DIMS)


# speedup vs baseline: 1.1627x; 1.0020x over previous
import functools

import jax
import jax.numpy as jnp
from jax import lax
from jax.experimental import pallas as pl
from jax.experimental.pallas import tpu as pltpu
from jax.experimental.pallas import tpu_sc as plsc

NUMS = 1000000
DIMS = 32
BATCH = 16384
HIST = 50
B = BATCH * HIST

NUM_WORKERS = 32
B_PER_W = B // NUM_WORKERS
NBUF = 4
CHUNK = 800
N_CHUNKS = B_PER_W // CHUNK
N_GROUPS = N_CHUNKS // NBUF


def _sc_gather(table, idx):
    mesh = plsc.VectorSubcoreMesh(core_axis_name="c", subcore_axis_name="s")

    @functools.partial(
        pl.kernel,
        mesh=mesh,
        out_type=jax.ShapeDtypeStruct((B, DIMS), jnp.float32),
        scratch_types=[
            pltpu.VMEM((B_PER_W,), jnp.int32),
            [pltpu.VMEM((CHUNK, DIMS), jnp.float32) for _ in range(NBUF)],
            [pltpu.SemaphoreType.DMA for _ in range(NBUF)],
        ],
        compiler_params=pltpu.CompilerParams(use_tc_tiling_on_sc=False),
    )
    def k(table_hbm, idx_hbm, out_hbm, idx_all, rows, sems):
        wid = lax.axis_index("s") * 2 + lax.axis_index("c")
        w_base = wid * B_PER_W

        pltpu.sync_copy(idx_hbm.at[pl.ds(w_base, B_PER_W)], idx_all)

        def gather_start(i, b):
            pltpu.async_copy(
                table_hbm.at[idx_all.at[pl.ds(i * CHUNK, CHUNK)]],
                rows[b],
                sems[b],
            )

        def gather_wait(i, b):
            pltpu.make_async_copy(
                table_hbm.at[idx_all.at[pl.ds(i * CHUNK, CHUNK)]],
                rows[b],
                sems[b],
            ).wait()

        def writeback(i, b):
            pltpu.sync_copy(
                rows[b], out_hbm.at[pl.ds(w_base + i * CHUNK, CHUNK)]
            )

        for b in range(NBUF):
            gather_start(b, b)

        def body(g, _):
            for b in range(NBUF):
                i = g * NBUF + b
                gather_wait(i, b)
                writeback(i, b)
                gather_start(i + NBUF, b)
            return 0

        lax.fori_loop(0, N_GROUPS - 1, body, 0)

        for b in range(NBUF):
            i = (N_GROUPS - 1) * NBUF + b
            gather_wait(i, b)
            writeback(i, b)

    return k(table, idx)


def kernel(x, table):
    idx = x.reshape(B)
    out = _sc_gather(table, idx)
    return out.reshape(BATCH, HIST, DIMS)
